# Initial kernel scaffold; baseline (speedup 1.0000x reference)
import jax
import jax.numpy as jnp
from jax.experimental import pallas as pl


def kernel(tokens, charges, aa_table, charge_table):
    raise NotImplementedError("write your pallas kernel here")


# speedup vs baseline: 5.4693x; 5.4693x over previous
import functools

import jax
import jax.numpy as jnp
from jax import lax
from jax.experimental import pallas as pl
from jax.experimental.pallas import tpu as pltpu
from jax.experimental.pallas import tpu_sc as plsc

DIM = 64
VOCAB_P2 = 1002
BATCH = 16384
SEQ = 50
N_ROWS = BATCH * (SEQ + 1)
NC, NS = 2, 16
NW = NC * NS
PER_W = N_ROWS // NW
SUBLEN = 128
GROUP = 512
SUB = GROUP // SUBLEN
NGRP = PER_W // GROUP
NBLK = N_ROWS // SUBLEN


def _gather_body(table_hbm, idx_hbm, out_hbm, idx_v, rows_v, sem):
    wid = lax.axis_index("s") * NC + lax.axis_index("c")
    base_blk = wid * (PER_W // SUBLEN)

    def body(g, carry):
        blk0 = base_blk + g * SUB
        pltpu.sync_copy(idx_hbm.at[pl.ds(blk0, SUB)], idx_v)
        copies = [
            pltpu.async_copy(table_hbm.at[idx_v.at[j]], rows_v.at[j], sem)
            for j in range(SUB)
        ]
        for c in copies:
            c.wait()
        pltpu.sync_copy(rows_v, out_hbm.at[pl.ds(blk0, SUB)])
        return carry

    lax.fori_loop(0, NGRP, body, 0)


_sc_gather = functools.partial(
    pl.kernel,
    out_type=jax.ShapeDtypeStruct((NBLK, SUBLEN, DIM), jnp.float32),
    mesh=plsc.VectorSubcoreMesh(core_axis_name="c", subcore_axis_name="s"),
    scratch_types=[
        pltpu.VMEM((SUB, SUBLEN), jnp.int32),
        pltpu.VMEM((SUB, SUBLEN, DIM), jnp.float32),
        pltpu.SemaphoreType.DMA,
    ],
    compiler_params=pltpu.CompilerParams(use_tc_tiling_on_sc=False),
)(_gather_body)


def kernel(tokens, charges, aa_table, charge_table):
    aa_table = aa_table.at[0].set(0.0)
    table = jnp.concatenate([aa_table, charge_table], axis=0)
    cidx = jnp.concatenate(
        [charges.astype(jnp.int32)[:, None] + VOCAB_P2, tokens.astype(jnp.int32)],
        axis=1,
    ).reshape(NBLK, SUBLEN)
    out = _sc_gather(table, cidx)
    return out.reshape(BATCH, SEQ + 1, DIM)


# speedup vs baseline: 5.5043x; 1.0064x over previous
import functools

import jax
import jax.numpy as jnp
from jax import lax
from jax.experimental import pallas as pl
from jax.experimental.pallas import tpu as pltpu
from jax.experimental.pallas import tpu_sc as plsc

DIM = 64
VOCAB_P2 = 1002
BATCH = 16384
SEQ = 50
N_ROWS = BATCH * (SEQ + 1)
NC, NS = 2, 16
NW = NC * NS
PER_W = N_ROWS // NW
SUBLEN = 128
SUB = 6
GROUP = SUB * SUBLEN
NGRP = PER_W // GROUP
NBLK = N_ROWS // SUBLEN


def _gather_body(table_hbm, idx_hbm, out_hbm, idx_v, rows_v, isem, gsem, ssem):
    wid = lax.axis_index("s") * NC + lax.axis_index("c")
    base_blk = wid * (PER_W // SUBLEN)

    pltpu.sync_copy(idx_hbm.at[pl.ds(base_blk, SUB)], idx_v.at[0])

    def body(i, carry):
        b = lax.rem(i, 2)
        pb = 1 - b
        blk0 = base_blk + i * SUB

        @pl.when(i >= 1)
        def _wait_idx():
            pltpu.make_async_copy(
                idx_hbm.at[pl.ds(base_blk, SUB)], idx_v.at[b], isem
            ).wait()

        gathers = [
            pltpu.async_copy(
                table_hbm.at[idx_v.at[b].at[j]], rows_v.at[b].at[j], gsem
            )
            for j in range(SUB)
        ]

        @pl.when(i < NGRP - 1)
        def _prefetch_idx():
            pltpu.async_copy(
                idx_hbm.at[pl.ds(blk0 + SUB, SUB)], idx_v.at[pb], isem
            )

        @pl.when(i >= 1)
        def _wait_store():
            pltpu.make_async_copy(
                rows_v.at[pb], out_hbm.at[pl.ds(base_blk, SUB)], ssem
            ).wait()

        for c in gathers:
            c.wait()
        pltpu.async_copy(rows_v.at[b], out_hbm.at[pl.ds(blk0, SUB)], ssem)
        return carry

    lax.fori_loop(0, NGRP, body, 0)
    lb = (NGRP - 1) % 2
    pltpu.make_async_copy(
        rows_v.at[lb], out_hbm.at[pl.ds(base_blk, SUB)], ssem
    ).wait()


_sc_gather = functools.partial(
    pl.kernel,
    out_type=jax.ShapeDtypeStruct((NBLK, SUBLEN, DIM), jnp.float32),
    mesh=plsc.VectorSubcoreMesh(core_axis_name="c", subcore_axis_name="s"),
    scratch_types=[
        pltpu.VMEM((2, SUB, SUBLEN), jnp.int32),
        pltpu.VMEM((2, SUB, SUBLEN, DIM), jnp.float32),
        pltpu.SemaphoreType.DMA,
        pltpu.SemaphoreType.DMA,
        pltpu.SemaphoreType.DMA,
    ],
    compiler_params=pltpu.CompilerParams(use_tc_tiling_on_sc=False),
)(_gather_body)


def kernel(tokens, charges, aa_table, charge_table):
    aa_table = aa_table.at[0].set(0.0)
    table = jnp.concatenate([aa_table, charge_table], axis=0)
    cidx = jnp.concatenate(
        [charges.astype(jnp.int32)[:, None] + VOCAB_P2, tokens.astype(jnp.int32)],
        axis=1,
    ).reshape(NBLK, SUBLEN)
    out = _sc_gather(table, cidx)
    return out.reshape(BATCH, SEQ + 1, DIM)
